# Initial kernel scaffold; baseline (speedup 1.0000x reference)
import jax
import jax.numpy as jnp
from jax.experimental import pallas as pl


def kernel(logits, labels, embeddings):
    raise NotImplementedError("write your pallas kernel here")


# speedup vs baseline: 2.6795x; 2.6795x over previous
import functools

import jax
import jax.numpy as jnp
from jax import lax
from jax.experimental import pallas as pl
from jax.experimental.pallas import tpu as pltpu

_S = 64.0
_MARGIN = 1.7

_R_BLOCK = 256
_C_BLOCK = 2048


def _acos_poly(x):
    p7 = -0.0012624911
    p6 = 0.0066700901
    p5 = -0.0170881256
    p4 = 0.0308918810
    p3 = -0.0501743046
    p2 = 0.0889789874
    p1 = -0.2145988016
    p0 = 1.5707963050
    r = p7
    for c in (p6, p5, p4, p3, p2, p1, p0):
        r = r * x + c
    return r * jnp.sqrt(jnp.maximum(1.0 - x, 0.0))


def _body(lab_ref, x_ref, o_ref):
    j = pl.program_id(1)
    lab = lab_ref[0, 0, :]
    local = lab - j * _C_BLOCK
    col = lax.broadcasted_iota(jnp.int32, (_R_BLOCK, _C_BLOCK), 1)
    hit = col == local[:, None]
    x = x_ref[...]
    t = jnp.sum(jnp.where(hit, x, 0.0), axis=1)
    m = _S * jnp.cos(_MARGIN * _acos_poly(t))
    o_ref[...] = jnp.where(hit, m[:, None], _S * x)


def kernel(logits, labels, embeddings):
    del embeddings
    rows, cols = logits.shape
    n_r = rows // _R_BLOCK
    n_c = pl.cdiv(cols, _C_BLOCK)
    lab3 = labels.astype(jnp.int32).reshape(n_r, 1, _R_BLOCK)
    return pl.pallas_call(
        _body,
        grid=(n_r, n_c),
        in_specs=[
            pl.BlockSpec((1, 1, _R_BLOCK), lambda i, j: (i, 0, 0)),
            pl.BlockSpec((_R_BLOCK, _C_BLOCK), lambda i, j: (i, j)),
        ],
        out_specs=pl.BlockSpec((_R_BLOCK, _C_BLOCK), lambda i, j: (i, j)),
        out_shape=jax.ShapeDtypeStruct((rows, cols), jnp.float32),
    )(lab3, logits)


# speedup vs baseline: 2.9020x; 1.0830x over previous
import functools

import jax
import jax.numpy as jnp
from jax import lax
from jax.experimental import pallas as pl
from jax.experimental.pallas import tpu as pltpu

_S = 64.0
_MARGIN = 1.7

_R_BLOCK = 1024
_C_BLOCK = 2048


def _acos_poly(x):
    p7 = -0.0012624911
    p6 = 0.0066700901
    p5 = -0.0170881256
    p4 = 0.0308918810
    p3 = -0.0501743046
    p2 = 0.0889789874
    p1 = -0.2145988016
    p0 = 1.5707963050
    r = p7
    for c in (p6, p5, p4, p3, p2, p1, p0):
        r = r * x + c
    return r * jnp.sqrt(jnp.maximum(1.0 - x, 0.0))


def _body(lab_ref, x_ref, o_ref):
    j = pl.program_id(1)
    lab = lab_ref[0, 0, :]
    local = lab - j * _C_BLOCK
    col = lax.broadcasted_iota(jnp.int32, (_R_BLOCK, _C_BLOCK), 1)
    hit = col == local[:, None]
    x = x_ref[...]
    t = jnp.sum(jnp.where(hit, x, 0.0), axis=1)
    m = _S * jnp.cos(_MARGIN * _acos_poly(t))
    o_ref[...] = jnp.where(hit, m[:, None], _S * x)


def kernel(logits, labels, embeddings):
    del embeddings
    rows, cols = logits.shape
    n_r = rows // _R_BLOCK
    n_c = pl.cdiv(cols, _C_BLOCK)
    lab3 = labels.astype(jnp.int32).reshape(n_r, 1, _R_BLOCK)
    return pl.pallas_call(
        _body,
        grid=(n_r, n_c),
        in_specs=[
            pl.BlockSpec((1, 1, _R_BLOCK), lambda i, j: (i, 0, 0)),
            pl.BlockSpec((_R_BLOCK, _C_BLOCK), lambda i, j: (i, j)),
        ],
        out_specs=pl.BlockSpec((_R_BLOCK, _C_BLOCK), lambda i, j: (i, j)),
        out_shape=jax.ShapeDtypeStruct((rows, cols), jnp.float32),
    )(lab3, logits)
